# Initial kernel scaffold; baseline (speedup 1.0000x reference)
import jax
import jax.numpy as jnp
from jax.experimental import pallas as pl


def kernel(x, edge_index, edge_attr, W1, b1, W2, b2, W3, b3, W4, b4, W5, b5, W6, b6, W7, b7, W8, b8):
    raise NotImplementedError("write your pallas kernel here")


# speedup vs baseline: 4.7000x; 4.7000x over previous
import functools

import jax
import jax.numpy as jnp
from jax import lax
from jax.experimental import pallas as pl
from jax.experimental.pallas import tpu as pltpu
from jax.experimental.pallas import tpu_sc as plsc

N_NODES = 10000
NP = 10240
D_H = 128
D_PAD_OUT = 128
K = 128
NC = 2
NS = 16
ROWS_PER_TILE = NP // NS


def _mesh():
    return plsc.VectorSubcoreMesh(
        core_axis_name="c", subcore_axis_name="s", num_cores=NC, num_subcores=NS
    )


def _make_agg(ep, d):
    ct = ep // (NC * NS * K)
    ept = ct * K
    epsc = NS * ept

    @functools.partial(
        pl.kernel,
        out_type=jax.ShapeDtypeStruct((NC, NP, d), jnp.float32),
        mesh=_mesh(),
        compiler_params=pltpu.CompilerParams(use_tc_tiling_on_sc=True),
        scratch_types=[
            pltpu.VMEM((K,), jnp.int32),
            pltpu.VMEM((K,), jnp.int32),
            pltpu.VMEM((K, 16), jnp.float32),
            pltpu.VMEM((K, d), jnp.float32),
            pltpu.VMEM_SHARED((NP, d), jnp.float32),
            pltpu.SemaphoreType.DMA,
        ],
    )
    def agg(g_hbm, src_hbm, dst_hbm, ewb_hbm, out_hbm,
            idx_v, dst_v, ew_v, rows_v, acc, sem):
        c = lax.axis_index("c")
        s = lax.axis_index("s")

        def zr(r, carry):
            for j in range(d // 16):
                rows_v[r, pl.ds(16 * j, 16)] = jnp.zeros((16,), jnp.float32)
            return carry
        lax.fori_loop(0, K, zr, None)
        for cp in range(ROWS_PER_TILE // K):
            pltpu.sync_copy(rows_v, acc.at[pl.ds(s * ROWS_PER_TILE + cp * K, K)])
        plsc.subcore_barrier()

        ebase = c * epsc + s * ept

        def chunk(ch, carry):
            base = ebase + ch * K
            pltpu.sync_copy(src_hbm.at[pl.ds(base, K)], idx_v)
            pltpu.sync_copy(dst_hbm.at[pl.ds(base, K)], dst_v)
            pltpu.sync_copy(ewb_hbm.at[pl.ds(base, K)], ew_v)
            pltpu.async_copy(g_hbm.at[idx_v], rows_v, sem).wait()

            def scale(kk, carry2):
                w = ew_v[kk, :]
                for j in range(d // 16):
                    sl = pl.ds(16 * j, 16)
                    rows_v[kk, sl] = rows_v[kk, sl] * w
                return carry2
            lax.fori_loop(0, K, scale, None)
            pltpu.sync_copy(rows_v, acc.at[dst_v], add=True)
            return carry
        lax.fori_loop(0, ct, chunk, None)

        plsc.subcore_barrier()
        pltpu.sync_copy(
            acc.at[pl.ds(s * ROWS_PER_TILE, ROWS_PER_TILE)],
            out_hbm.at[c, pl.ds(s * ROWS_PER_TILE, ROWS_PER_TILE)],
        )

    return agg


BM = 1280


def _dot(y, w):
    return jnp.dot(y, w, preferred_element_type=jnp.float32)


def _t_first(xp, w1, deg0, deg1):
    def body(x_ref, w_ref, d0_ref, d1_ref, g_ref, dv_ref):
        degs = d0_ref[:, 0:1] + d1_ref[:, 0:1] + 1.0
        dinv = jnp.where(degs > 0, 1.0 / jnp.sqrt(degs), 0.0)
        g_ref[...] = _dot(x_ref[...], w_ref[...]) * dinv
        dv_ref[...] = jnp.broadcast_to(dinv, (BM, D_H))

    return pl.pallas_call(
        body,
        grid=(NP // BM,),
        in_specs=[
            pl.BlockSpec((BM, D_H), lambda i: (i, 0)),
            pl.BlockSpec((D_H, D_H), lambda i: (0, 0)),
            pl.BlockSpec((BM, D_H), lambda i: (i, 0)),
            pl.BlockSpec((BM, D_H), lambda i: (i, 0)),
        ],
        out_specs=[pl.BlockSpec((BM, D_H), lambda i: (i, 0))] * 2,
        out_shape=[jax.ShapeDtypeStruct((NP, D_H), jnp.float32)] * 2,
    )(xp, w1, deg0, deg1)


def _t_mid(a0, a1, g_prev, dvb, bvec, w, dn):
    def body(a0_ref, a1_ref, g_ref, dv_ref, b_ref, w_ref, o_ref):
        dv = dv_ref[...]
        z = dv * (a0_ref[...] + a1_ref[...] + g_ref[...]) + b_ref[...]
        o_ref[...] = _dot(jnp.maximum(z, 0.0), w_ref[...]) * dv

    return pl.pallas_call(
        body,
        grid=(NP // BM,),
        in_specs=[
            pl.BlockSpec((BM, D_H), lambda i: (i, 0)),
            pl.BlockSpec((BM, D_H), lambda i: (i, 0)),
            pl.BlockSpec((BM, D_H), lambda i: (i, 0)),
            pl.BlockSpec((BM, D_H), lambda i: (i, 0)),
            pl.BlockSpec((1, D_H), lambda i: (0, 0)),
            pl.BlockSpec((D_H, dn), lambda i: (0, 0)),
        ],
        out_specs=pl.BlockSpec((BM, dn), lambda i: (i, 0)),
        out_shape=jax.ShapeDtypeStruct((NP, dn), jnp.float32),
    )(a0, a1, g_prev, dvb, bvec, w)


def _t_last(a0, a1, g8, dvb, bvec):
    def body(a0_ref, a1_ref, g_ref, dv_ref, b_ref, o_ref):
        o_ref[...] = dv_ref[...] * (a0_ref[...] + a1_ref[...] + g_ref[...]) \
            + b_ref[...]

    w16 = pl.BlockSpec((BM, D_PAD_OUT), lambda i: (i, 0))
    return pl.pallas_call(
        body,
        grid=(NP // BM,),
        in_specs=[
            w16, w16, w16,
            pl.BlockSpec((BM, D_PAD_OUT), lambda i: (i, 0)),
            pl.BlockSpec((1, D_PAD_OUT), lambda i: (0, 0)),
        ],
        out_specs=w16,
        out_shape=jax.ShapeDtypeStruct((NP, D_PAD_OUT), jnp.float32),
    )(a0, a1, g8, dvb, bvec)


def kernel(x, edge_index, edge_attr, W1, b1, W2, b2, W3, b3, W4, b4,
           W5, b5, W6, b6, W7, b7, W8, b8):
    e = edge_index.shape[1]
    ep = -(-e // (NC * NS * K)) * (NC * NS * K)
    pad_e = ep - e

    src = jnp.pad(edge_index[0], (0, pad_e))
    dst = jnp.pad(edge_index[1], (0, pad_e))
    ewp = jnp.pad(edge_attr, (0, pad_e))
    ewb = jnp.repeat(ewp[:, None], 16, axis=1)
    xp = jnp.pad(x, ((0, NP - x.shape[0]), (0, 0)))
    w8p = jnp.pad(W8, ((0, 0), (0, D_PAD_OUT - W8.shape[1])))
    b8p = jnp.pad(b8, (0, D_PAD_OUT - b8.shape[0]))

    agg_fn = _make_agg(ep, D_H)

    ones = jnp.ones((NP, D_H), jnp.float32)
    degs = agg_fn(ones, src, dst, ewb)
    g, dvb = _t_first(xp, W1, degs[0], degs[1])

    weights = [W2, W3, W4, W5, W6, W7, w8p]
    biases = [b1, b2, b3, b4, b5, b6, b7]
    for i in range(7):
        a = agg_fn(g, src, dst, ewb)
        dn = D_PAD_OUT if i == 6 else D_H
        g = _t_mid(a[0], a[1], g, dvb,
                   biases[i].reshape(1, D_H), weights[i], dn)

    a8 = agg_fn(g, src, dst, ewb)
    out16 = _t_last(a8[0], a8[1], g, dvb[:, :D_PAD_OUT],
                    b8p.reshape(1, D_PAD_OUT))
    return out16[:N_NODES, :2]


# speedup vs baseline: 4.9733x; 1.0581x over previous
import functools

import jax
import jax.numpy as jnp
from jax import lax
from jax.experimental import pallas as pl
from jax.experimental.pallas import tpu as pltpu
from jax.experimental.pallas import tpu_sc as plsc

N_NODES = 10000
NP = 10240
D_H = 128
D_PAD_OUT = 128
K = 128
NC = 2
NS = 16
ROWS_PER_TILE = NP // NS


def _mesh():
    return plsc.VectorSubcoreMesh(
        core_axis_name="c", subcore_axis_name="s", num_cores=NC, num_subcores=NS
    )


def _make_agg(ep, d):
    ct = ep // (NC * NS * K)
    ept = ct * K
    epsc = NS * ept

    @functools.partial(
        pl.kernel,
        out_type=jax.ShapeDtypeStruct((NC, NP, d), jnp.float32),
        mesh=_mesh(),
        compiler_params=pltpu.CompilerParams(use_tc_tiling_on_sc=True),
        scratch_types=[
            pltpu.VMEM((2, K), jnp.int32),
            pltpu.VMEM((2, K), jnp.int32),
            pltpu.VMEM((2, K // 8, 128), jnp.float32),
            pltpu.VMEM((2, K, d), jnp.float32),
            pltpu.VMEM_SHARED((NP, d), jnp.float32),
            pltpu.SemaphoreType.DMA((2,)),
            pltpu.SemaphoreType.DMA((2,)),
        ],
    )
    def agg(g_hbm, src_hbm, dst_hbm, ewb_hbm, out_hbm,
            idx2, dst2, ew2, rows2, acc, sems, semg):
        c = lax.axis_index("c")
        s = lax.axis_index("s")
        ebase = c * epsc + s * ept

        def start_small(ch, p):
            base = pl.multiple_of(ebase + ch * K, K)
            base8 = pl.multiple_of((ebase + ch * K) // 8, K // 8)
            pltpu.async_copy(src_hbm.at[pl.ds(base, K)], idx2.at[p], sems.at[p])
            pltpu.async_copy(dst_hbm.at[pl.ds(base, K)], dst2.at[p], sems.at[p])
            pltpu.async_copy(ewb_hbm.at[pl.ds(base8, K // 8)], ew2.at[p],
                             sems.at[p])

        def wait_small(p):
            pltpu.make_async_copy(src_hbm.at[pl.ds(0, K)], idx2.at[p],
                                  sems.at[p]).wait()
            pltpu.make_async_copy(dst_hbm.at[pl.ds(0, K)], dst2.at[p],
                                  sems.at[p]).wait()
            pltpu.make_async_copy(ewb_hbm.at[pl.ds(0, K // 8)], ew2.at[p],
                                  sems.at[p]).wait()

        def start_gather(p):
            pltpu.async_copy(g_hbm.at[idx2.at[p]], rows2.at[p], semg.at[p])

        def wait_gather(p):
            pltpu.make_async_copy(g_hbm.at[idx2.at[p]], rows2.at[p],
                                  semg.at[p]).wait()

        start_small(0, 0)
        zbuf = rows2.at[0]

        def zr(r, carry):
            for j in range(d // 16):
                zbuf[r, pl.ds(16 * j, 16)] = jnp.zeros((16,), jnp.float32)
            return carry
        lax.fori_loop(0, K, zr, None)
        for cp in range(ROWS_PER_TILE // K):
            zoff = pl.multiple_of(s * ROWS_PER_TILE + cp * K, K)
            pltpu.sync_copy(zbuf, acc.at[pl.ds(zoff, K)])
        plsc.subcore_barrier()

        wait_small(0)
        start_gather(0)
        start_small(1, 1)

        @pl.loop(0, ct, step=2)
        def it(c0):
            for b in range(2):
                ch = c0 + b
                q = 1 - b
                if b == 0:
                    wait_small(q)
                    start_gather(q)
                else:
                    @pl.when(c0 + 2 < ct)
                    def _():
                        wait_small(q)
                        start_gather(q)
                wait_gather(b)
                rp = rows2.at[b]
                wp = ew2.at[b]

                @plsc.parallel_loop(0, K // 8)
                def scale(r):
                    for u in range(8):
                        w = wp[r, pl.ds(16 * u, 16)]
                        for j in range(d // 16):
                            sl = pl.ds(16 * j, 16)
                            rp[r * 8 + u, sl] = rp[r * 8 + u, sl] * w
                pltpu.sync_copy(rp, acc.at[dst2.at[b]], add=True)

                @pl.when(ch + 2 < ct)
                def _():
                    start_small(ch + 2, b)

        plsc.subcore_barrier()
        woff = pl.multiple_of(s * ROWS_PER_TILE, ROWS_PER_TILE)
        pltpu.sync_copy(
            acc.at[pl.ds(woff, ROWS_PER_TILE)],
            out_hbm.at[c, pl.ds(woff, ROWS_PER_TILE)],
        )

    return agg


BM = 1280


def _dot(y, w):
    return jnp.dot(y, w, preferred_element_type=jnp.float32)


def _t_first(xp, w1, deg0, deg1):
    def body(x_ref, w_ref, d0_ref, d1_ref, g_ref, dv_ref):
        degs = d0_ref[:, 0:1] + d1_ref[:, 0:1] + 1.0
        dinv = jnp.where(degs > 0, 1.0 / jnp.sqrt(degs), 0.0)
        g_ref[...] = _dot(x_ref[...], w_ref[...]) * dinv
        dv_ref[...] = jnp.broadcast_to(dinv, (BM, D_H))

    return pl.pallas_call(
        body,
        grid=(NP // BM,),
        in_specs=[
            pl.BlockSpec((BM, D_H), lambda i: (i, 0)),
            pl.BlockSpec((D_H, D_H), lambda i: (0, 0)),
            pl.BlockSpec((BM, D_H), lambda i: (i, 0)),
            pl.BlockSpec((BM, D_H), lambda i: (i, 0)),
        ],
        out_specs=[pl.BlockSpec((BM, D_H), lambda i: (i, 0))] * 2,
        out_shape=[jax.ShapeDtypeStruct((NP, D_H), jnp.float32)] * 2,
    )(xp, w1, deg0, deg1)


def _t_mid(a0, a1, g_prev, dvb, bvec, w, dn):
    def body(a0_ref, a1_ref, g_ref, dv_ref, b_ref, w_ref, o_ref):
        dv = dv_ref[...]
        z = dv * (a0_ref[...] + a1_ref[...] + g_ref[...]) + b_ref[...]
        o_ref[...] = _dot(jnp.maximum(z, 0.0), w_ref[...]) * dv

    return pl.pallas_call(
        body,
        grid=(NP // BM,),
        in_specs=[
            pl.BlockSpec((BM, D_H), lambda i: (i, 0)),
            pl.BlockSpec((BM, D_H), lambda i: (i, 0)),
            pl.BlockSpec((BM, D_H), lambda i: (i, 0)),
            pl.BlockSpec((BM, D_H), lambda i: (i, 0)),
            pl.BlockSpec((1, D_H), lambda i: (0, 0)),
            pl.BlockSpec((D_H, dn), lambda i: (0, 0)),
        ],
        out_specs=pl.BlockSpec((BM, dn), lambda i: (i, 0)),
        out_shape=jax.ShapeDtypeStruct((NP, dn), jnp.float32),
    )(a0, a1, g_prev, dvb, bvec, w)


def _t_last(a0, a1, g8, dvb, bvec):
    def body(a0_ref, a1_ref, g_ref, dv_ref, b_ref, o_ref):
        o_ref[...] = dv_ref[...] * (a0_ref[...] + a1_ref[...] + g_ref[...]) \
            + b_ref[...]

    w16 = pl.BlockSpec((BM, D_PAD_OUT), lambda i: (i, 0))
    return pl.pallas_call(
        body,
        grid=(NP // BM,),
        in_specs=[
            w16, w16, w16,
            pl.BlockSpec((BM, D_PAD_OUT), lambda i: (i, 0)),
            pl.BlockSpec((1, D_PAD_OUT), lambda i: (0, 0)),
        ],
        out_specs=w16,
        out_shape=jax.ShapeDtypeStruct((NP, D_PAD_OUT), jnp.float32),
    )(a0, a1, g8, dvb, bvec)


def kernel(x, edge_index, edge_attr, W1, b1, W2, b2, W3, b3, W4, b4,
           W5, b5, W6, b6, W7, b7, W8, b8):
    e = edge_index.shape[1]
    ep = -(-e // (NC * NS * K * 2)) * (NC * NS * K * 2)
    pad_e = ep - e

    src = jnp.pad(edge_index[0], (0, pad_e))
    dst = jnp.pad(edge_index[1], (0, pad_e))
    ewp = jnp.pad(edge_attr, (0, pad_e))
    ewb = jnp.repeat(ewp[:, None], 16, axis=1).reshape(ep // 8, 128)
    xp = jnp.pad(x, ((0, NP - x.shape[0]), (0, 0)))
    w8p = jnp.pad(W8, ((0, 0), (0, D_PAD_OUT - W8.shape[1])))
    b8p = jnp.pad(b8, (0, D_PAD_OUT - b8.shape[0]))

    agg_fn = _make_agg(ep, D_H)

    ones = jnp.ones((NP, D_H), jnp.float32)
    degs = agg_fn(ones, src, dst, ewb)
    g, dvb = _t_first(xp, W1, degs[0], degs[1])

    weights = [W2, W3, W4, W5, W6, W7, w8p]
    biases = [b1, b2, b3, b4, b5, b6, b7]
    for i in range(7):
        a = agg_fn(g, src, dst, ewb)
        dn = D_PAD_OUT if i == 6 else D_H
        g = _t_mid(a[0], a[1], g, dvb,
                   biases[i].reshape(1, D_H), weights[i], dn)

    a8 = agg_fn(g, src, dst, ewb)
    out16 = _t_last(a8[0], a8[1], g, dvb[:, :D_PAD_OUT],
                    b8p.reshape(1, D_PAD_OUT))
    return out16[:N_NODES, :2]


# speedup vs baseline: 9.0806x; 1.8259x over previous
import functools

import jax
import jax.numpy as jnp
from jax import lax
from jax.experimental import pallas as pl
from jax.experimental.pallas import tpu as pltpu
from jax.experimental.pallas import tpu_sc as plsc

N_NODES = 10000
NP = 10240
D_H = 128
DH2 = D_H // 2
K = 128
NC = 2
NS = 16
ROWS_PER_TILE = NP // NS


def _mesh():
    return plsc.VectorSubcoreMesh(
        core_axis_name="c", subcore_axis_name="s", num_cores=NC, num_subcores=NS
    )


def _make_agg(ep):
    ept = ep // NS
    ct = ept // K

    @functools.partial(
        pl.kernel,
        out_type=jax.ShapeDtypeStruct((NC, NP, DH2), jnp.float32),
        mesh=_mesh(),
        compiler_params=pltpu.CompilerParams(use_tc_tiling_on_sc=False),
        scratch_types=[
            pltpu.VMEM((2, K), jnp.int32),
            pltpu.VMEM((2, K), jnp.int32),
            pltpu.VMEM((2, K // 8, 128), jnp.float32),
            pltpu.VMEM((2, K, DH2), jnp.float32),
            pltpu.VMEM_SHARED((NP, DH2), jnp.float32),
            pltpu.VMEM_SHARED((NP, DH2), jnp.float32),
            pltpu.SemaphoreType.DMA((2,)),
            pltpu.SemaphoreType.DMA((2,)),
        ],
    )
    def agg(g_hbm, src_hbm, dst_hbm, ewb_hbm, out_hbm,
            idx2, dst2, ew2, rows2, g_sp, acc, sems, semg):
        c = lax.axis_index("c")
        s = lax.axis_index("s")
        ebase = s * ept

        def start_small(ch, p):
            base = pl.multiple_of(ebase + ch * K, K)
            base8 = pl.multiple_of((ebase + ch * K) // 8, K // 8)
            pltpu.async_copy(src_hbm.at[pl.ds(base, K)], idx2.at[p], sems.at[p])
            pltpu.async_copy(dst_hbm.at[pl.ds(base, K)], dst2.at[p], sems.at[p])
            pltpu.async_copy(ewb_hbm.at[pl.ds(base8, K // 8)], ew2.at[p],
                             sems.at[p])

        def wait_small(p):
            pltpu.make_async_copy(src_hbm.at[pl.ds(0, K)], idx2.at[p],
                                  sems.at[p]).wait()
            pltpu.make_async_copy(dst_hbm.at[pl.ds(0, K)], dst2.at[p],
                                  sems.at[p]).wait()
            pltpu.make_async_copy(ewb_hbm.at[pl.ds(0, K // 8)], ew2.at[p],
                                  sems.at[p]).wait()

        def start_gather(p):
            pltpu.async_copy(g_sp.at[idx2.at[p]], rows2.at[p], semg.at[p])

        def wait_gather(p):
            pltpu.make_async_copy(g_sp.at[idx2.at[p]], rows2.at[p],
                                  semg.at[p]).wait()

        start_small(0, 0)
        soff = pl.multiple_of(s * ROWS_PER_TILE, ROWS_PER_TILE)
        pltpu.sync_copy(g_hbm.at[c, pl.ds(soff, ROWS_PER_TILE)],
                        g_sp.at[pl.ds(soff, ROWS_PER_TILE)])
        zbuf = rows2.at[0]

        def zr(r, carry):
            for j in range(DH2 // 16):
                zbuf[r, pl.ds(16 * j, 16)] = jnp.zeros((16,), jnp.float32)
            return carry
        lax.fori_loop(0, K, zr, None)
        for cp in range(ROWS_PER_TILE // K):
            zoff = pl.multiple_of(s * ROWS_PER_TILE + cp * K, K)
            pltpu.sync_copy(zbuf, acc.at[pl.ds(zoff, K)])
        plsc.subcore_barrier()

        wait_small(0)
        start_gather(0)
        start_small(1, 1)

        @pl.loop(0, ct, step=2)
        def it(c0):
            for b in range(2):
                ch = c0 + b
                q = 1 - b
                if b == 0:
                    wait_small(q)
                    start_gather(q)
                else:
                    @pl.when(c0 + 2 < ct)
                    def _():
                        wait_small(q)
                        start_gather(q)
                wait_gather(b)
                rp = rows2.at[b]
                wp = ew2.at[b]

                @plsc.parallel_loop(0, K // 8)
                def scale(r):
                    for u in range(8):
                        w = wp[r, pl.ds(16 * u, 16)]
                        for j in range(DH2 // 16):
                            sl = pl.ds(16 * j, 16)
                            rp[r * 8 + u, sl] = rp[r * 8 + u, sl] * w
                pltpu.sync_copy(rp, acc.at[dst2.at[b]], add=True)

                @pl.when(ch + 2 < ct)
                def _():
                    start_small(ch + 2, b)

        plsc.subcore_barrier()
        pltpu.sync_copy(
            acc.at[pl.ds(soff, ROWS_PER_TILE)],
            out_hbm.at[c, pl.ds(soff, ROWS_PER_TILE)],
        )

    return agg


BM = 1280


def _dot(y, w):
    return jnp.dot(y, w, preferred_element_type=jnp.float32)


_SPEC2 = pl.BlockSpec((NC, BM, DH2), lambda i: (0, i, 0))
_SPECF = pl.BlockSpec((BM, D_H), lambda i: (i, 0))


def _cat(ref):
    return jnp.concatenate([ref[0], ref[1]], axis=1)


def _split_store(o_ref, val):
    o_ref[0] = val[:, :DH2]
    o_ref[1] = val[:, DH2:]


def _t_first(xp, w1, degs):
    def body(x_ref, w_ref, dg_ref, g_ref, dv_ref):
        deg = dg_ref[0, :, 0:1] + 1.0
        dinv = jnp.where(deg > 0, 1.0 / jnp.sqrt(deg), 0.0)
        g = _dot(x_ref[...], w_ref[...]) * dinv
        _split_store(g_ref, g)
        dv_ref[...] = jnp.broadcast_to(dinv, (BM, D_H))

    return pl.pallas_call(
        body,
        grid=(NP // BM,),
        in_specs=[
            _SPECF,
            pl.BlockSpec((D_H, D_H), lambda i: (0, 0)),
            _SPEC2,
        ],
        out_specs=[_SPEC2, _SPECF],
        out_shape=[jax.ShapeDtypeStruct((NC, NP, DH2), jnp.float32),
                   jax.ShapeDtypeStruct((NP, D_H), jnp.float32)],
    )(xp, w1, degs)


def _t_mid(a, g_prev, dvb, bvec, w):
    def body(a_ref, g_ref, dv_ref, b_ref, w_ref, o_ref):
        dv = dv_ref[...]
        z = dv * (_cat(a_ref) + _cat(g_ref)) + b_ref[...]
        _split_store(o_ref, _dot(jnp.maximum(z, 0.0), w_ref[...]) * dv)

    return pl.pallas_call(
        body,
        grid=(NP // BM,),
        in_specs=[
            _SPEC2, _SPEC2, _SPECF,
            pl.BlockSpec((1, D_H), lambda i: (0, 0)),
            pl.BlockSpec((D_H, D_H), lambda i: (0, 0)),
        ],
        out_specs=_SPEC2,
        out_shape=jax.ShapeDtypeStruct((NC, NP, DH2), jnp.float32),
    )(a, g_prev, dvb, bvec, w)


def _t_last(a, g8, dvb, bvec):
    def body(a_ref, g_ref, dv_ref, b_ref, o_ref):
        o_ref[...] = dv_ref[...] * (_cat(a_ref) + _cat(g_ref)) + b_ref[...]

    return pl.pallas_call(
        body,
        grid=(NP // BM,),
        in_specs=[
            _SPEC2, _SPEC2, _SPECF,
            pl.BlockSpec((1, D_H), lambda i: (0, 0)),
        ],
        out_specs=_SPECF,
        out_shape=jax.ShapeDtypeStruct((NP, D_H), jnp.float32),
    )(a, g8, dvb, bvec)


def kernel(x, edge_index, edge_attr, W1, b1, W2, b2, W3, b3, W4, b4,
           W5, b5, W6, b6, W7, b7, W8, b8):
    e = edge_index.shape[1]
    ep = -(-e // (NS * K * 2)) * (NS * K * 2)
    pad_e = ep - e

    src = jnp.pad(edge_index[0], (0, pad_e))
    dst = jnp.pad(edge_index[1], (0, pad_e))
    ewp = jnp.pad(edge_attr, (0, pad_e))
    ewb = jnp.repeat(ewp[:, None], 16, axis=1).reshape(ep // 8, 128)
    xp = jnp.pad(x, ((0, NP - x.shape[0]), (0, 0)))
    w8p = jnp.pad(W8, ((0, 0), (0, D_H - W8.shape[1])))
    b8p = jnp.pad(b8, (0, D_H - b8.shape[0]))

    agg_fn = _make_agg(ep)

    ones = jnp.ones((NC, NP, DH2), jnp.float32)
    degs = agg_fn(ones, src, dst, ewb)
    g, dvb = _t_first(xp, W1, degs)

    weights = [W2, W3, W4, W5, W6, W7, w8p]
    biases = [b1, b2, b3, b4, b5, b6, b7]
    for i in range(7):
        a = agg_fn(g, src, dst, ewb)
        g = _t_mid(a, g, dvb, biases[i].reshape(1, D_H), weights[i])

    a8 = agg_fn(g, src, dst, ewb)
    out = _t_last(a8, g, dvb, b8p.reshape(1, D_H))
    return out[:N_NODES, :2]


# speedup vs baseline: 10.3100x; 1.1354x over previous
import functools

import jax
import jax.numpy as jnp
from jax import lax
from jax.experimental import pallas as pl
from jax.experimental.pallas import tpu as pltpu
from jax.experimental.pallas import tpu_sc as plsc

N_NODES = 10000
NP = 10240
D_H = 128
DH2 = D_H // 2
K = 256
NC = 2
NS = 16
ROWS_PER_TILE = NP // NS


def _mesh():
    return plsc.VectorSubcoreMesh(
        core_axis_name="c", subcore_axis_name="s", num_cores=NC, num_subcores=NS
    )


def _make_agg(ep):
    ept = ep // NS
    ct = ept // K

    @functools.partial(
        pl.kernel,
        out_type=jax.ShapeDtypeStruct((NC, NP, DH2), jnp.float32),
        mesh=_mesh(),
        compiler_params=pltpu.CompilerParams(use_tc_tiling_on_sc=False),
        scratch_types=[
            pltpu.VMEM((2, 2, 2, 128), jnp.int32),
            pltpu.VMEM((2, K // 8, 128), jnp.float32),
            pltpu.VMEM((2, K, DH2), jnp.float32),
            pltpu.VMEM_SHARED((NP, DH2), jnp.float32),
            pltpu.VMEM_SHARED((NP, DH2), jnp.float32),
            pltpu.SemaphoreType.DMA((2,)),
            pltpu.SemaphoreType.DMA((2,)),
        ],
    )
    def agg(g_hbm, sd_hbm, ewb_hbm, out_hbm,
            sd2, ew2, rows2, g_sp, acc, sems, semg):
        c = lax.axis_index("c")
        s = lax.axis_index("s")
        ebase = s * ept

        def start_small(ch, p):
            b128 = pl.multiple_of((ebase + ch * K) // 128, 2)
            base8 = pl.multiple_of((ebase + ch * K) // 8, K // 8)
            pltpu.async_copy(sd_hbm.at[:, pl.ds(b128, 2)], sd2.at[p],
                             sems.at[p])
            pltpu.async_copy(ewb_hbm.at[pl.ds(base8, K // 8)], ew2.at[p],
                             sems.at[p])

        def wait_small(p):
            pltpu.make_async_copy(sd_hbm.at[:, pl.ds(0, 2)], sd2.at[p],
                                  sems.at[p]).wait()
            pltpu.make_async_copy(ewb_hbm.at[pl.ds(0, K // 8)], ew2.at[p],
                                  sems.at[p]).wait()

        def start_gather(p):
            for h in range(2):
                pltpu.async_copy(g_sp.at[sd2.at[p, 0, h]],
                                 rows2.at[p, pl.ds(h * 128, 128)], semg.at[p])

        def wait_gather(p):
            for h in range(2):
                pltpu.make_async_copy(g_sp.at[sd2.at[p, 0, h]],
                                      rows2.at[p, pl.ds(h * 128, 128)],
                                      semg.at[p]).wait()

        start_small(0, 0)
        soff = pl.multiple_of(s * ROWS_PER_TILE, ROWS_PER_TILE)
        pltpu.sync_copy(g_hbm.at[c, pl.ds(soff, ROWS_PER_TILE)],
                        g_sp.at[pl.ds(soff, ROWS_PER_TILE)])
        zbuf = rows2.at[0]

        def zr(r, carry):
            for j in range(DH2 // 16):
                zbuf[r, pl.ds(16 * j, 16)] = jnp.zeros((16,), jnp.float32)
            return carry
        lax.fori_loop(0, 128, zr, None)
        for cp in range(ROWS_PER_TILE // 128):
            zoff = pl.multiple_of(s * ROWS_PER_TILE + cp * 128, 128)
            pltpu.sync_copy(rows2.at[0, pl.ds(0, 128)],
                            acc.at[pl.ds(zoff, 128)])
        plsc.subcore_barrier()

        wait_small(0)
        start_gather(0)
        start_small(1, 1)

        @pl.loop(0, ct, step=2)
        def it(c0):
            for b in range(2):
                ch = c0 + b
                q = 1 - b
                if b == 0:
                    wait_small(q)
                    start_gather(q)
                else:
                    @pl.when(c0 + 2 < ct)
                    def _():
                        wait_small(q)
                        start_gather(q)
                wait_gather(b)
                rp = rows2.at[b]
                wp = ew2.at[b]

                @plsc.parallel_loop(0, K // 8)
                def scale(r):
                    for u in range(8):
                        w = wp[r, pl.ds(16 * u, 16)]
                        for j in range(DH2 // 16):
                            sl = pl.ds(16 * j, 16)
                            rp[r * 8 + u, sl] = rp[r * 8 + u, sl] * w
                for h in range(2):
                    pltpu.sync_copy(rows2.at[b, pl.ds(h * 128, 128)],
                                    acc.at[sd2.at[b, 1, h]], add=True)

                @pl.when(ch + 2 < ct)
                def _():
                    start_small(ch + 2, b)

        plsc.subcore_barrier()
        pltpu.sync_copy(
            acc.at[pl.ds(soff, ROWS_PER_TILE)],
            out_hbm.at[c, pl.ds(soff, ROWS_PER_TILE)],
        )

    return agg


BM = 1280


def _dot(y, w):
    return jnp.dot(y, w, preferred_element_type=jnp.float32)


_SPEC2 = pl.BlockSpec((NC, BM, DH2), lambda i: (0, i, 0))
_SPECF = pl.BlockSpec((BM, D_H), lambda i: (i, 0))


def _cat(ref):
    return jnp.concatenate([ref[0], ref[1]], axis=1)


def _split_store(o_ref, val):
    o_ref[0] = val[:, :DH2]
    o_ref[1] = val[:, DH2:]


def _t_first(xp, w1, degs):
    def body(x_ref, w_ref, dg_ref, g_ref, dv_ref):
        deg = dg_ref[0, :, 0:1] + 1.0
        dinv = jnp.where(deg > 0, 1.0 / jnp.sqrt(deg), 0.0)
        g = _dot(x_ref[...], w_ref[...]) * dinv
        _split_store(g_ref, g)
        dv_ref[...] = jnp.broadcast_to(dinv, (BM, D_H))

    return pl.pallas_call(
        body,
        grid=(NP // BM,),
        in_specs=[
            _SPECF,
            pl.BlockSpec((D_H, D_H), lambda i: (0, 0)),
            _SPEC2,
        ],
        out_specs=[_SPEC2, _SPECF],
        out_shape=[jax.ShapeDtypeStruct((NC, NP, DH2), jnp.float32),
                   jax.ShapeDtypeStruct((NP, D_H), jnp.float32)],
    )(xp, w1, degs)


def _t_mid(a, g_prev, dvb, bvec, w):
    def body(a_ref, g_ref, dv_ref, b_ref, w_ref, o_ref):
        dv = dv_ref[...]
        z = dv * (_cat(a_ref) + _cat(g_ref)) + b_ref[...]
        _split_store(o_ref, _dot(jnp.maximum(z, 0.0), w_ref[...]) * dv)

    return pl.pallas_call(
        body,
        grid=(NP // BM,),
        in_specs=[
            _SPEC2, _SPEC2, _SPECF,
            pl.BlockSpec((1, D_H), lambda i: (0, 0)),
            pl.BlockSpec((D_H, D_H), lambda i: (0, 0)),
        ],
        out_specs=_SPEC2,
        out_shape=jax.ShapeDtypeStruct((NC, NP, DH2), jnp.float32),
    )(a, g_prev, dvb, bvec, w)


def _t_last(a, g8, dvb, bvec):
    def body(a_ref, g_ref, dv_ref, b_ref, o_ref):
        o_ref[...] = dv_ref[...] * (_cat(a_ref) + _cat(g_ref)) + b_ref[...]

    return pl.pallas_call(
        body,
        grid=(NP // BM,),
        in_specs=[
            _SPEC2, _SPEC2, _SPECF,
            pl.BlockSpec((1, D_H), lambda i: (0, 0)),
        ],
        out_specs=_SPECF,
        out_shape=jax.ShapeDtypeStruct((NP, D_H), jnp.float32),
    )(a, g8, dvb, bvec)


def kernel(x, edge_index, edge_attr, W1, b1, W2, b2, W3, b3, W4, b4,
           W5, b5, W6, b6, W7, b7, W8, b8):
    e = edge_index.shape[1]
    ep = -(-e // (NS * K * 2)) * (NS * K * 2)
    pad_e = ep - e

    sd = jnp.pad(edge_index, ((0, 0), (0, pad_e))).reshape(2, ep // 128, 128)
    ewp = jnp.pad(edge_attr, (0, pad_e))
    ewb = jnp.repeat(ewp[:, None], 16, axis=1).reshape(ep // 8, 128)
    xp = jnp.pad(x, ((0, NP - x.shape[0]), (0, 0)))
    w8p = jnp.pad(W8, ((0, 0), (0, D_H - W8.shape[1])))
    b8p = jnp.pad(b8, (0, D_H - b8.shape[0]))

    agg_fn = _make_agg(ep)

    ones = jnp.ones((NC, NP, DH2), jnp.float32)
    degs = agg_fn(ones, sd, ewb)
    g, dvb = _t_first(xp, W1, degs)

    weights = [W2, W3, W4, W5, W6, W7, w8p]
    biases = [b1, b2, b3, b4, b5, b6, b7]
    for i in range(7):
        a = agg_fn(g, sd, ewb)
        g = _t_mid(a, g, dvb, biases[i].reshape(1, D_H), weights[i])

    a8 = agg_fn(g, sd, ewb)
    out = _t_last(a8, g, dvb, b8p.reshape(1, D_H))
    return out[:N_NODES, :2]


# speedup vs baseline: 10.9509x; 1.0622x over previous
import functools

import jax
import jax.numpy as jnp
from jax import lax
from jax.experimental import pallas as pl
from jax.experimental.pallas import tpu as pltpu
from jax.experimental.pallas import tpu_sc as plsc

N_NODES = 10000
NP = 10240
D_H = 128
DH2 = D_H // 2
K = 256
NC = 2
NS = 16
ROWS_PER_TILE = NP // NS


def _mesh():
    return plsc.VectorSubcoreMesh(
        core_axis_name="c", subcore_axis_name="s", num_cores=NC, num_subcores=NS
    )


def _make_agg(ep):
    ept = ep // NS
    ct = ept // K

    @functools.partial(
        pl.kernel,
        out_type=jax.ShapeDtypeStruct((NC, NP, DH2), jnp.float32),
        mesh=_mesh(),
        compiler_params=pltpu.CompilerParams(use_tc_tiling_on_sc=False),
        scratch_types=[
            pltpu.VMEM((2, 2, 2, 128), jnp.int32),
            pltpu.VMEM((2, K // 8, 128), jnp.float32),
            pltpu.VMEM((2, K, DH2), jnp.float32),
            pltpu.VMEM_SHARED((NP, DH2), jnp.float32),
            pltpu.VMEM_SHARED((NP, DH2), jnp.float32),
            pltpu.SemaphoreType.DMA((2,)),
            pltpu.SemaphoreType.DMA((2,)),
        ],
    )
    def agg(g_hbm, sd_hbm, ewb_hbm, out_hbm,
            sd2, ew2, rows2, g_sp, acc, sems, semg):
        c = lax.axis_index("c")
        s = lax.axis_index("s")
        ebase = s * ept

        def start_small(ch, p):
            b128 = pl.multiple_of((ebase + ch * K) // 128, 2)
            base8 = pl.multiple_of((ebase + ch * K) // 8, K // 8)
            pltpu.async_copy(sd_hbm.at[:, pl.ds(b128, 2)], sd2.at[p],
                             sems.at[p])
            pltpu.async_copy(ewb_hbm.at[pl.ds(base8, K // 8)], ew2.at[p],
                             sems.at[p])

        def wait_small(p):
            pltpu.make_async_copy(sd_hbm.at[:, pl.ds(0, 2)], sd2.at[p],
                                  sems.at[p]).wait()
            pltpu.make_async_copy(ewb_hbm.at[pl.ds(0, K // 8)], ew2.at[p],
                                  sems.at[p]).wait()

        def start_gather(p):
            for h in range(2):
                pltpu.async_copy(g_sp.at[sd2.at[p, 0, h]],
                                 rows2.at[p, pl.ds(h * 128, 128)], semg.at[p])

        def wait_gather(p):
            for h in range(2):
                pltpu.make_async_copy(g_sp.at[sd2.at[p, 0, h]],
                                      rows2.at[p, pl.ds(h * 128, 128)],
                                      semg.at[p]).wait()

        start_small(0, 0)
        soff = pl.multiple_of(s * ROWS_PER_TILE, ROWS_PER_TILE)
        pltpu.sync_copy(g_hbm.at[c, pl.ds(soff, ROWS_PER_TILE)],
                        g_sp.at[pl.ds(soff, ROWS_PER_TILE)])
        zbuf = rows2.at[0]

        def zr(r, carry):
            for j in range(DH2 // 16):
                zbuf[r, pl.ds(16 * j, 16)] = jnp.zeros((16,), jnp.float32)
            return carry
        lax.fori_loop(0, 128, zr, None)
        for cp in range(ROWS_PER_TILE // 128):
            zoff = pl.multiple_of(s * ROWS_PER_TILE + cp * 128, 128)
            pltpu.sync_copy(rows2.at[0, pl.ds(0, 128)],
                            acc.at[pl.ds(zoff, 128)])
        plsc.subcore_barrier()

        wait_small(0)
        start_gather(0)
        start_small(1, 1)

        @pl.loop(0, ct, step=2)
        def it(c0):
            for b in range(2):
                ch = c0 + b
                q = 1 - b
                if b == 0:
                    wait_small(q)
                    start_gather(q)
                else:
                    @pl.when(c0 + 2 < ct)
                    def _():
                        wait_small(q)
                        start_gather(q)
                wait_gather(b)
                rp = rows2.at[b]
                wp = ew2.at[b]

                @plsc.parallel_loop(0, K // 8)
                def scale(r):
                    for u in range(8):
                        w = wp[r, pl.ds(16 * u, 16)]
                        for j in range(DH2 // 16):
                            sl = pl.ds(16 * j, 16)
                            rp[r * 8 + u, sl] = rp[r * 8 + u, sl] * w
                for h in range(2):
                    pltpu.sync_copy(rows2.at[b, pl.ds(h * 128, 128)],
                                    acc.at[sd2.at[b, 1, h]], add=True)

                @pl.when(ch + 2 < ct)
                def _():
                    start_small(ch + 2, b)

        plsc.subcore_barrier()
        pltpu.sync_copy(
            acc.at[pl.ds(soff, ROWS_PER_TILE)],
            out_hbm.at[c, pl.ds(soff, ROWS_PER_TILE)],
        )

    return agg


def _make_deg(ep):
    ept = ep // (NC * NS)
    ct = ept // K
    epsc = NS * ept

    @functools.partial(
        pl.kernel,
        out_type=jax.ShapeDtypeStruct((NC, NP, DH2), jnp.float32),
        mesh=_mesh(),
        compiler_params=pltpu.CompilerParams(use_tc_tiling_on_sc=False),
        scratch_types=[
            pltpu.VMEM((2, 2, 2, 128), jnp.int32),
            pltpu.VMEM((2, K // 8, 128), jnp.float32),
            pltpu.VMEM((2, K, DH2), jnp.float32),
            pltpu.VMEM_SHARED((NP, DH2), jnp.float32),
            pltpu.SemaphoreType.DMA((2,)),
        ],
    )
    def deg(sd_hbm, ewb_hbm, out_hbm, sd2, ew2, rows2, acc, sems):
        c = lax.axis_index("c")
        s = lax.axis_index("s")
        ebase = c * epsc + s * ept

        def start_small(ch, p):
            b128 = pl.multiple_of((ebase + ch * K) // 128, 2)
            base8 = pl.multiple_of((ebase + ch * K) // 8, K // 8)
            pltpu.async_copy(sd_hbm.at[:, pl.ds(b128, 2)], sd2.at[p],
                             sems.at[p])
            pltpu.async_copy(ewb_hbm.at[pl.ds(base8, K // 8)], ew2.at[p],
                             sems.at[p])

        def wait_small(p):
            pltpu.make_async_copy(sd_hbm.at[:, pl.ds(0, 2)], sd2.at[p],
                                  sems.at[p]).wait()
            pltpu.make_async_copy(ewb_hbm.at[pl.ds(0, K // 8)], ew2.at[p],
                                  sems.at[p]).wait()

        start_small(0, 0)
        start_small(1, 1)
        soff = pl.multiple_of(s * ROWS_PER_TILE, ROWS_PER_TILE)
        zbuf = rows2.at[0]

        def zr(r, carry):
            for j in range(DH2 // 16):
                zbuf[r, pl.ds(16 * j, 16)] = jnp.zeros((16,), jnp.float32)
            return carry
        lax.fori_loop(0, 128, zr, None)
        for cp in range(ROWS_PER_TILE // 128):
            zoff = pl.multiple_of(s * ROWS_PER_TILE + cp * 128, 128)
            pltpu.sync_copy(rows2.at[0, pl.ds(0, 128)],
                            acc.at[pl.ds(zoff, 128)])
        plsc.subcore_barrier()

        @pl.loop(0, ct, step=2)
        def it(c0):
            for b in range(2):
                ch = c0 + b
                wait_small(b)
                rp = rows2.at[b]
                wp = ew2.at[b]

                @plsc.parallel_loop(0, K // 8)
                def build(r):
                    for u in range(8):
                        w = wp[r, pl.ds(16 * u, 16)]
                        for j in range(DH2 // 16):
                            rp[r * 8 + u, pl.ds(16 * j, 16)] = w
                for h in range(2):
                    pltpu.sync_copy(rows2.at[b, pl.ds(h * 128, 128)],
                                    acc.at[sd2.at[b, 1, h]], add=True)

                @pl.when(ch + 2 < ct)
                def _():
                    start_small(ch + 2, b)

        plsc.subcore_barrier()
        pltpu.sync_copy(
            acc.at[pl.ds(soff, ROWS_PER_TILE)],
            out_hbm.at[c, pl.ds(soff, ROWS_PER_TILE)],
        )

    return deg


BM = 1280


def _dot(y, w):
    return jnp.dot(y, w, preferred_element_type=jnp.float32)


_SPEC2 = pl.BlockSpec((NC, BM, DH2), lambda i: (0, i, 0))
_SPECF = pl.BlockSpec((BM, D_H), lambda i: (i, 0))


def _cat(ref):
    return jnp.concatenate([ref[0], ref[1]], axis=1)


def _split_store(o_ref, val):
    o_ref[0] = val[:, :DH2]
    o_ref[1] = val[:, DH2:]


def _t_first(xp, w1, degs):
    def body(x_ref, w_ref, dg_ref, g_ref, dv_ref):
        deg = dg_ref[0, :, 0:1] + dg_ref[1, :, 0:1] + 1.0
        dinv = jnp.where(deg > 0, 1.0 / jnp.sqrt(deg), 0.0)
        g = _dot(x_ref[...], w_ref[...]) * dinv
        _split_store(g_ref, g)
        dv_ref[...] = jnp.broadcast_to(dinv, (BM, D_H))

    return pl.pallas_call(
        body,
        grid=(NP // BM,),
        in_specs=[
            _SPECF,
            pl.BlockSpec((D_H, D_H), lambda i: (0, 0)),
            _SPEC2,
        ],
        out_specs=[_SPEC2, _SPECF],
        out_shape=[jax.ShapeDtypeStruct((NC, NP, DH2), jnp.float32),
                   jax.ShapeDtypeStruct((NP, D_H), jnp.float32)],
    )(xp, w1, degs)


def _t_mid(a, g_prev, dvb, bvec, w):
    def body(a_ref, g_ref, dv_ref, b_ref, w_ref, o_ref):
        dv = dv_ref[...]
        z = dv * (_cat(a_ref) + _cat(g_ref)) + b_ref[...]
        _split_store(o_ref, _dot(jnp.maximum(z, 0.0), w_ref[...]) * dv)

    return pl.pallas_call(
        body,
        grid=(NP // BM,),
        in_specs=[
            _SPEC2, _SPEC2, _SPECF,
            pl.BlockSpec((1, D_H), lambda i: (0, 0)),
            pl.BlockSpec((D_H, D_H), lambda i: (0, 0)),
        ],
        out_specs=_SPEC2,
        out_shape=jax.ShapeDtypeStruct((NC, NP, DH2), jnp.float32),
    )(a, g_prev, dvb, bvec, w)


def _t_last(a, g8, dvb, bvec):
    def body(a_ref, g_ref, dv_ref, b_ref, o_ref):
        o_ref[...] = dv_ref[...] * (_cat(a_ref) + _cat(g_ref)) + b_ref[...]

    return pl.pallas_call(
        body,
        grid=(NP // BM,),
        in_specs=[
            _SPEC2, _SPEC2, _SPECF,
            pl.BlockSpec((1, D_H), lambda i: (0, 0)),
        ],
        out_specs=_SPECF,
        out_shape=jax.ShapeDtypeStruct((NP, D_H), jnp.float32),
    )(a, g8, dvb, bvec)


def kernel(x, edge_index, edge_attr, W1, b1, W2, b2, W3, b3, W4, b4,
           W5, b5, W6, b6, W7, b7, W8, b8):
    e = edge_index.shape[1]
    ep = -(-e // (NS * K * 2)) * (NS * K * 2)
    pad_e = ep - e

    sd = jnp.pad(edge_index, ((0, 0), (0, pad_e))).reshape(2, ep // 128, 128)
    ewp = jnp.pad(edge_attr, (0, pad_e))
    ewb = jnp.repeat(ewp[:, None], 16, axis=1).reshape(ep // 8, 128)
    xp = jnp.pad(x, ((0, NP - x.shape[0]), (0, 0)))
    w8p = jnp.pad(W8, ((0, 0), (0, D_H - W8.shape[1])))
    b8p = jnp.pad(b8, (0, D_H - b8.shape[0]))

    agg_fn = _make_agg(ep)

    degs = _make_deg(ep)(sd, ewb)
    g, dvb = _t_first(xp, W1, degs)

    weights = [W2, W3, W4, W5, W6, W7, w8p]
    biases = [b1, b2, b3, b4, b5, b6, b7]
    for i in range(7):
        a = agg_fn(g, sd, ewb)
        g = _t_mid(a, g, dvb, biases[i].reshape(1, D_H), weights[i])

    a8 = agg_fn(g, sd, ewb)
    out = _t_last(a8, g, dvb, b8p.reshape(1, D_H))
    return out[:N_NODES, :2]
